# Initial kernel scaffold; baseline (speedup 1.0000x reference)
import jax
import jax.numpy as jnp
from jax.experimental import pallas as pl


def kernel(hidden_states, gate_w, gate_up_proj, down_proj):
    raise NotImplementedError("write your pallas kernel here")


# speedup vs baseline: 2.0227x; 2.0227x over previous
import functools

import jax
import jax.numpy as jnp
from jax.experimental import pallas as pl

NUM_EXPERTS = 8
TOP_K = 2
EPAD = 128


def _router_body(x_ref, gw_ref, s_ref):
    x = x_ref[...]
    logits = jnp.dot(x, gw_ref[...], preferred_element_type=jnp.float32)
    col = jax.lax.broadcasted_iota(jnp.int32, logits.shape, 1)
    valid = col < NUM_EXPERTS
    l = jnp.where(valid, logits, -1e30)
    m = jnp.max(l, axis=1, keepdims=True)
    ex = jnp.where(valid, jnp.exp(l - m), 0.0)
    p = ex / jnp.sum(ex, axis=1, keepdims=True)
    m1 = jnp.max(l, axis=1, keepdims=True)
    i1 = jnp.min(jnp.where(l == m1, col, EPAD), axis=1, keepdims=True)
    l2 = jnp.where(col == i1, -1e30, l)
    m2 = jnp.max(l2, axis=1, keepdims=True)
    i2 = jnp.min(jnp.where(l2 == m2, col, EPAD), axis=1, keepdims=True)
    w1 = jnp.sum(jnp.where(col == i1, p, 0.0), axis=1, keepdims=True)
    w2 = jnp.sum(jnp.where(col == i2, p, 0.0), axis=1, keepdims=True)
    s = w1 + w2
    s_ref[...] = jnp.where(col == i1, w1 / s, 0.0) + jnp.where(col == i2, w2 / s, 0.0)


def _moe_body(x_ref, w1_ref, w2_ref, s_ref, o_ref, *, d_ff):
    e = pl.program_id(1)
    x = x_ref[...]
    gu = jnp.dot(x, w1_ref[0], preferred_element_type=jnp.float32)
    g = gu[:, :d_ff]
    u = gu[:, d_ff:]
    h = u * (g * jax.nn.sigmoid(g))
    y = jnp.dot(h, w2_ref[0], preferred_element_type=jnp.float32)
    col = jax.lax.broadcasted_iota(jnp.int32, s_ref.shape, 1)
    c = jnp.sum(jnp.where(col == e, s_ref[...], 0.0), axis=1, keepdims=True)
    contrib = y * c

    @pl.when(e == 0)
    def _():
        o_ref[...] = contrib

    @pl.when(e > 0)
    def _():
        o_ref[...] += contrib


@jax.jit
def kernel(hidden_states, gate_w, gate_up_proj, down_proj):
    B, S, D = hidden_states.shape
    bs = B * S
    d_ff = down_proj.shape[1]
    x = hidden_states.reshape(bs, D)
    gw_pad = jnp.zeros((D, EPAD), jnp.float32).at[:, :NUM_EXPERTS].set(gate_w)

    scattered = pl.pallas_call(
        _router_body,
        out_shape=jax.ShapeDtypeStruct((bs, EPAD), jnp.float32),
    )(x, gw_pad)

    TB = 2
    tb = bs // TB
    out = pl.pallas_call(
        functools.partial(_moe_body, d_ff=d_ff),
        grid=(TB, NUM_EXPERTS),
        in_specs=[
            pl.BlockSpec((tb, D), lambda t, e: (t, 0)),
            pl.BlockSpec((1, D, 2 * d_ff), lambda t, e: (e, 0, 0)),
            pl.BlockSpec((1, d_ff, D), lambda t, e: (e, 0, 0)),
            pl.BlockSpec((tb, EPAD), lambda t, e: (t, 0)),
        ],
        out_specs=pl.BlockSpec((tb, D), lambda t, e: (t, 0)),
        out_shape=jax.ShapeDtypeStruct((bs, D), jnp.float32),
    )(x, gate_up_proj, down_proj, scattered)
    return out.reshape(B, S, D)
